# baseline (speedup 1.0000x reference)
import functools
import math

import jax
import jax.numpy as jnp
from jax import lax
from jax.experimental import pallas as pl
from jax.experimental.pallas import tpu as pltpu
from jax.experimental.pallas import tpu_sc as plsc

_NUM_WORKERS = 32
_CHUNK = 128


def _emb_kernel(idx_hbm, table_hbm, out_hbm, idx_v, rows0, rows1, sem0, sem1):
    n_chunks = idx_hbm.shape[1]
    wid = lax.axis_index("s") * 2 + lax.axis_index("c")
    base = wid * (n_chunks * _CHUNK)

    pltpu.sync_copy(idx_hbm.at[wid], idx_v)

    scale = jnp.full((16,), 8.0, dtype=jnp.float32)

    def scale_rows(rows):
        def srow(r, carry):
            for c in range(4):
                rows[r, pl.ds(c * 16, 16)] = rows[r, pl.ds(c * 16, 16)] * scale
            return carry
        lax.fori_loop(0, _CHUNK, srow, 0, unroll=2)

    pltpu.async_copy(table_hbm.at[idx_v.at[0]], rows0, sem0)

    def body(j, carry):
        del carry
        buf = j % 2

        @pl.when(jnp.logical_and(buf == 0, j + 1 < n_chunks))
        def _():
            pltpu.async_copy(table_hbm.at[idx_v.at[j + 1]], rows1, sem1)

        @pl.when(jnp.logical_and(buf == 1, j + 1 < n_chunks))
        def _():
            pltpu.async_copy(table_hbm.at[idx_v.at[j + 1]], rows0, sem0)

        @pl.when(buf == 0)
        def _():
            pltpu.make_async_copy(table_hbm.at[idx_v.at[j]], rows0, sem0).wait()
            scale_rows(rows0)
            pltpu.sync_copy(rows0, out_hbm.at[pl.ds(base + j * _CHUNK, _CHUNK)])

        @pl.when(buf == 1)
        def _():
            pltpu.make_async_copy(table_hbm.at[idx_v.at[j]], rows1, sem1).wait()
            scale_rows(rows1)
            pltpu.sync_copy(rows1, out_hbm.at[pl.ds(base + j * _CHUNK, _CHUNK)])

        return 0

    lax.fori_loop(0, n_chunks, body, 0)


def kernel(tokens, weight):
    B, T = tokens.shape
    V, D = weight.shape
    n = B * T
    assert n % (_NUM_WORKERS * _CHUNK) == 0
    n_chunks = n // (_NUM_WORKERS * _CHUNK)

    idx = tokens.reshape(_NUM_WORKERS, n_chunks, _CHUNK).astype(jnp.int32)

    mesh = plsc.VectorSubcoreMesh(core_axis_name="c", subcore_axis_name="s")
    out = pl.kernel(
        _emb_kernel,
        out_type=jax.ShapeDtypeStruct((n, D), jnp.float32),
        mesh=mesh,
        scratch_types=[
            pltpu.VMEM((n_chunks, _CHUNK), jnp.int32),
            pltpu.VMEM((_CHUNK, D), jnp.float32),
            pltpu.VMEM((_CHUNK, D), jnp.float32),
            pltpu.SemaphoreType.DMA,
            pltpu.SemaphoreType.DMA,
        ],
        compiler_params=pltpu.CompilerParams(use_tc_tiling_on_sc=False),
    )(idx, weight)
    return out.reshape(B, T, D)
